# Initial kernel scaffold; baseline (speedup 1.0000x reference)
import jax
import jax.numpy as jnp
from jax.experimental import pallas as pl


def kernel(f_atoms, f_bonds, a2b, b2a, b2revb, W_i, W_h, W_o, b_o):
    raise NotImplementedError("write your pallas kernel here")


# speedup vs baseline: 1.0472x; 1.0472x over previous
import functools

import jax
import jax.numpy as jnp
from jax import lax
from jax.experimental import pallas as pl
from jax.experimental.pallas import tpu as pltpu
from jax.experimental.pallas import tpu_sc as plsc

N_ATOMS = 10000
MAX_NB = 32
N_BONDS = 320000
ATOM_FDIM = 128
BOND_FDIM = 16
HIDDEN = 128
DEPTH = 4

NC = 2
NS = 16
NW = NC * NS

A_PAD = 10240
A_PER_W = A_PAD // NW
A_CHUNK = 4
N_ACH = A_PER_W // A_CHUNK

B_PER_W = N_BONDS // NW
B_CHUNK = 80
N_BCH = B_PER_W // B_CHUNK

_LANES = 8


def _worker_id():
    return lax.axis_index("s") * NC + lax.axis_index("c")


def _sc_gather_sum_body(msg_hbm, a2b_hbm, agg_hbm, idx_v, rows_v, out_v, sem):
    wid = _worker_id()

    def chunk(c, carry):
        ibase = wid * (A_PER_W * MAX_NB) + c * (A_CHUNK * MAX_NB)
        pltpu.sync_copy(a2b_hbm.at[pl.ds(ibase, A_CHUNK * MAX_NB)], idx_v)
        pltpu.async_copy(msg_hbm.at[idx_v], rows_v, sem).wait()
        for g in range(A_CHUNK):
            def red(j, acc):
                r = g * MAX_NB + j
                return tuple(
                    acc[k] + jnp.maximum(rows_v[r, pl.ds(16 * k, 16)], 0.0)
                    for k in range(_LANES))
            acc = lax.fori_loop(
                0, MAX_NB, red,
                tuple(jnp.zeros((16,), jnp.float32) for _ in range(_LANES)))
            for k in range(_LANES):
                out_v[g, pl.ds(16 * k, 16)] = acc[k]
        obase = wid * A_PER_W + c * A_CHUNK
        pltpu.sync_copy(out_v, agg_hbm.at[pl.ds(obase, A_CHUNK)])
        return carry

    lax.fori_loop(0, N_ACH, chunk, 0)


def _sc_gather_sum(msg, a2b_flat):
    fn = pl.kernel(
        _sc_gather_sum_body,
        out_type=jax.ShapeDtypeStruct((A_PAD, HIDDEN), jnp.float32),
        mesh=plsc.VectorSubcoreMesh(
            core_axis_name="c", subcore_axis_name="s",
            num_cores=NC, num_subcores=NS),
        scratch_types=[
            pltpu.VMEM((A_CHUNK * MAX_NB,), jnp.int32),
            pltpu.VMEM((A_CHUNK * MAX_NB, HIDDEN), jnp.float32),
            pltpu.VMEM((A_CHUNK, HIDDEN), jnp.float32),
            pltpu.SemaphoreType.DMA,
        ],
    )
    return fn(msg, a2b_flat)


def _sc_edge_body(msg_hbm, agg_hbm, b2a_hbm, b2revb_hbm, pre_hbm,
                  ia_v, ir_v, rows_a, rows_r, out_v, sem):
    wid = _worker_id()

    def chunk(c, carry):
        base = wid * B_PER_W + c * B_CHUNK
        pltpu.sync_copy(b2a_hbm.at[pl.ds(base, B_CHUNK)], ia_v)
        pltpu.sync_copy(b2revb_hbm.at[pl.ds(base, B_CHUNK)], ir_v)
        ca = pltpu.async_copy(agg_hbm.at[ia_v], rows_a, sem)
        cr = pltpu.async_copy(msg_hbm.at[ir_v], rows_r, sem)
        ca.wait()
        cr.wait()

        def row(r, inner):
            for k in range(_LANES):
                out_v[r, pl.ds(16 * k, 16)] = (
                    rows_a[r, pl.ds(16 * k, 16)]
                    - jnp.maximum(rows_r[r, pl.ds(16 * k, 16)], 0.0))
            return inner

        lax.fori_loop(0, B_CHUNK, row, 0)
        pltpu.sync_copy(out_v, pre_hbm.at[pl.ds(base, B_CHUNK)])
        return carry

    lax.fori_loop(0, N_BCH, chunk, 0)


def _sc_edge(msg, agg, b2a, b2revb):
    fn = pl.kernel(
        _sc_edge_body,
        out_type=jax.ShapeDtypeStruct((N_BONDS, HIDDEN), jnp.float32),
        mesh=plsc.VectorSubcoreMesh(
            core_axis_name="c", subcore_axis_name="s",
            num_cores=NC, num_subcores=NS),
        scratch_types=[
            pltpu.VMEM((B_CHUNK,), jnp.int32),
            pltpu.VMEM((B_CHUNK,), jnp.int32),
            pltpu.VMEM((B_CHUNK, HIDDEN), jnp.float32),
            pltpu.VMEM((B_CHUNK, HIDDEN), jnp.float32),
            pltpu.VMEM((B_CHUNK, HIDDEN), jnp.float32),
            pltpu.SemaphoreType.DMA,
        ],
    )
    return fn(msg, agg, b2a, b2revb)


_MM_ROWS = 2000


def _tc_mm_body(x_ref, w_ref, o_ref):
    o_ref[...] = jnp.dot(x_ref[...], w_ref[...],
                         preferred_element_type=jnp.float32)


def _tc_mm(x, w):
    m, k = x.shape
    n = w.shape[1]
    grid = m // _MM_ROWS
    return pl.pallas_call(
        _tc_mm_body,
        grid=(grid,),
        in_specs=[
            pl.BlockSpec((_MM_ROWS, k), lambda i: (i, 0)),
            pl.BlockSpec((k, n), lambda i: (0, 0)),
        ],
        out_specs=pl.BlockSpec((_MM_ROWS, n), lambda i: (i, 0)),
        out_shape=jax.ShapeDtypeStruct((m, n), jnp.float32),
    )(x, w)


def _tc_update_body(pre_ref, inp_ref, w_ref, o_ref):
    o_ref[...] = jnp.maximum(
        inp_ref[...] + jnp.dot(pre_ref[...], w_ref[...],
                               preferred_element_type=jnp.float32), 0.0)


def _tc_update(pre, inp, w):
    m = pre.shape[0]
    grid = m // _MM_ROWS
    return pl.pallas_call(
        _tc_update_body,
        grid=(grid,),
        in_specs=[
            pl.BlockSpec((_MM_ROWS, HIDDEN), lambda i: (i, 0)),
            pl.BlockSpec((_MM_ROWS, HIDDEN), lambda i: (i, 0)),
            pl.BlockSpec((HIDDEN, HIDDEN), lambda i: (0, 0)),
        ],
        out_specs=pl.BlockSpec((_MM_ROWS, HIDDEN), lambda i: (i, 0)),
        out_shape=jax.ShapeDtypeStruct((m, HIDDEN), jnp.float32),
    )(pre, inp, w)


_OUT_ROWS = 2000


def _tc_out_body(fa_ref, am_ref, wa_ref, wm_ref, bo_ref, o_ref):
    acc = jnp.dot(fa_ref[...], wa_ref[...], preferred_element_type=jnp.float32)
    acc = acc + jnp.dot(am_ref[...], wm_ref[...],
                        preferred_element_type=jnp.float32)
    o_ref[...] = jnp.maximum(acc + bo_ref[...], 0.0)


def _tc_out(f_atoms, a_msg, w_a, w_m, b_o):
    m = f_atoms.shape[0]
    grid = m // _OUT_ROWS
    return pl.pallas_call(
        _tc_out_body,
        grid=(grid,),
        in_specs=[
            pl.BlockSpec((_OUT_ROWS, ATOM_FDIM), lambda i: (i, 0)),
            pl.BlockSpec((_OUT_ROWS, HIDDEN), lambda i: (i, 0)),
            pl.BlockSpec((ATOM_FDIM, HIDDEN), lambda i: (0, 0)),
            pl.BlockSpec((HIDDEN, HIDDEN), lambda i: (0, 0)),
            pl.BlockSpec((1, HIDDEN), lambda i: (0, 0)),
        ],
        out_specs=pl.BlockSpec((_OUT_ROWS, HIDDEN), lambda i: (i, 0)),
        out_shape=jax.ShapeDtypeStruct((m, HIDDEN), jnp.float32),
    )(f_atoms, a_msg, w_a, w_m, b_o)


def kernel(f_atoms, f_bonds, a2b, b2a, b2revb, W_i, W_h, W_o, b_o):
    a2b_flat = jnp.pad(a2b, ((0, A_PAD - N_ATOMS), (0, 0))).reshape(-1)
    inp = _tc_mm(f_bonds, W_i)
    msg = inp
    for _ in range(DEPTH - 1):
        agg = _sc_gather_sum(msg, a2b_flat)
        pre = _sc_edge(msg, agg, b2a, b2revb)
        msg = _tc_update(pre, inp, W_h)
    a_msg = _sc_gather_sum(msg, a2b_flat)[:N_ATOMS]
    return _tc_out(f_atoms, a_msg, W_o[:ATOM_FDIM], W_o[ATOM_FDIM:],
                   b_o.reshape(1, HIDDEN))
